# baseline (speedup 1.0000x reference)
import functools

import jax
import jax.numpy as jnp
from jax import lax
from jax.experimental import pallas as pl
from jax.experimental.pallas import tpu as pltpu
from jax.experimental.pallas import tpu_sc as plsc

_B = 16384
_D = 64
_NF = 14
_OUT_W = (_NF + 1) * _D
_NW = 32
_CB = 128
_NIT = _B // _CB // _NW
_L = 16
_TCB = 256


def _gather_body(sid_hbm, table_hbm, out_hbm, idx_v, gbuf, blk_v,
                 psem, gsem, osem):
    c_ax = lax.axis_index("c")
    s_ax = lax.axis_index("s")
    w = s_ax * 2 + c_ax

    pro = []
    for j in range(_NIT):
        cj = w + _NW * j
        pro.append(pltpu.async_copy(sid_hbm.at[pl.ds(cj * _CB, _CB)],
                                    idx_v.at[j], psem))
    for cp in pro:
        cp.wait()

    def fire_gather(j, jp):
        pltpu.async_copy(table_hbm.at[idx_v.at[j]], gbuf.at[jp], gsem.at[jp])

    def out_slice(i):
        return out_hbm.at[:, pl.ds((w + _NW * i) * _CB, _CB)]

    fire_gather(0, 0)
    iota = lax.iota(jnp.int32, _L)

    for i in range(_NIT):
        p = i % 2
        blk = blk_v.at[p]
        if i + 1 < _NIT:
            fire_gather(i + 1, 1 - p)
        if i >= 2:
            pltpu.make_async_copy(blk, out_slice(i - 2), osem.at[p]).wait()
        pltpu.make_async_copy(table_hbm.at[idx_v.at[i]], gbuf.at[p],
                              gsem.at[p]).wait()

        def row(r, carry, p=p, blk=blk):
            cols = jnp.broadcast_to(r, (_L,)).astype(jnp.int32)
            for q in range(_D // _L):
                val = gbuf[p, r, pl.ds(q * _L, _L)]
                plsc.store_scatter(blk, [iota + (q * _L), cols], val)
            return carry

        lax.fori_loop(0, _CB, row, 0)
        pltpu.async_copy(blk, out_slice(i), osem.at[p])

    for j in (_NIT - 2, _NIT - 1):
        pltpu.make_async_copy(blk_v.at[j % 2], out_slice(j),
                              osem.at[j % 2]).wait()


@jax.jit
def _gather_sc(sid, table128):
    kern = pl.kernel(
        _gather_body,
        out_type=jax.ShapeDtypeStruct((_D, _B), jnp.float32),
        mesh=plsc.VectorSubcoreMesh(core_axis_name="c", subcore_axis_name="s"),
        compiler_params=pltpu.CompilerParams(needs_layout_passes=False),
        scratch_types=[
            pltpu.VMEM((_NIT, _CB), jnp.int32),
            pltpu.VMEM((2, _CB, 2 * _D), jnp.float32),
            pltpu.VMEM((2, _D, _CB), jnp.float32),
            pltpu.SemaphoreType.DMA,
            pltpu.SemaphoreType.DMA((2,)),
            pltpu.SemaphoreType.DMA((2,)),
        ],
    )
    return kern(sid, table128)


def _assemble_body(base_ref, feats_ref, w_ref, b_ref, out_ref):
    out_ref[pl.ds(0, _D), :] = base_ref[...]
    for f in range(_NF):
        fv = feats_ref[f, :]
        wf = w_ref[f, :]
        bf = b_ref[f, :]
        out_ref[pl.ds((f + 1) * _D, _D), :] = (
            wf[:, None] * fv[None, :] + bf[:, None])


@jax.jit
def _assemble_tc(base2, feats, wmat, bmat):
    return pl.pallas_call(
        _assemble_body,
        grid=(_B // _TCB,),
        in_specs=[
            pl.BlockSpec((_D, _TCB), lambda j: (0, j)),
            pl.BlockSpec((_NF, _TCB), lambda j: (0, j)),
            pl.BlockSpec((_NF, _D), lambda j: (0, 0)),
            pl.BlockSpec((_NF, _D), lambda j: (0, 0)),
        ],
        out_specs=pl.BlockSpec((_OUT_W, _TCB), lambda j: (0, j)),
        out_shape=jax.ShapeDtypeStruct((_OUT_W, _B), jnp.float32),
    )(base2, feats, wmat, bmat)


def kernel(student_id, table,
           feat_age, feat_gender, feat_ethnicity, feat_location, feat_gpa,
           feat_test_scores, feat_courses, feat_major, feat_attendance,
           feat_participation, feat_feedback, feat_study_habits,
           feat_social_activity, feat_stress_level,
           W_age, W_gender, W_ethnicity, W_location, W_gpa,
           W_test_scores, W_courses, W_major, W_attendance,
           W_participation, W_feedback, W_study_habits,
           W_social_activity, W_stress_level,
           b_age, b_gender, b_ethnicity, b_location, b_gpa,
           b_test_scores, b_courses, b_major, b_attendance,
           b_participation, b_feedback, b_study_habits,
           b_social_activity, b_stress_level):
    feats = jnp.stack([
        feat_age, feat_gender, feat_ethnicity, feat_location, feat_gpa,
        feat_test_scores, feat_courses, feat_major, feat_attendance,
        feat_participation, feat_feedback, feat_study_habits,
        feat_social_activity, feat_stress_level])
    wmat = jnp.concatenate([
        W_age, W_gender, W_ethnicity, W_location, W_gpa,
        W_test_scores, W_courses, W_major, W_attendance,
        W_participation, W_feedback, W_study_habits,
        W_social_activity, W_stress_level], axis=0)
    bmat = jnp.stack([
        b_age, b_gender, b_ethnicity, b_location, b_gpa,
        b_test_scores, b_courses, b_major, b_attendance,
        b_participation, b_feedback, b_study_habits,
        b_social_activity, b_stress_level])
    table128 = jnp.concatenate(
        [table, jnp.zeros((table.shape[0], _D), table.dtype)], axis=1)
    base2 = _gather_sc(student_id, table128)
    out2 = _assemble_tc(base2, feats, wmat, bmat)
    return out2.T


# speedup vs baseline: 1.2039x; 1.2039x over previous
import functools

import jax
import jax.numpy as jnp
from jax import lax
from jax.experimental import pallas as pl
from jax.experimental.pallas import tpu as pltpu
from jax.experimental.pallas import tpu_sc as plsc

_B = 16384
_D = 64
_NF = 14
_OUT_W = (_NF + 1) * _D
_NW = 32
_CB = 128
_NIT = _B // _CB // _NW
_L = 16
_TCB = 512


def _gather_body(sid_hbm, table_hbm, out_hbm, idx_v, gbuf, blk_v,
                 psem, gsem, osem):
    c_ax = lax.axis_index("c")
    s_ax = lax.axis_index("s")
    w = s_ax * 2 + c_ax

    pro = []
    for j in range(_NIT):
        cj = w + _NW * j
        pro.append(pltpu.async_copy(sid_hbm.at[pl.ds(cj * _CB, _CB)],
                                    idx_v.at[j], psem))
    for cp in pro:
        cp.wait()

    def out_slice(i):
        return out_hbm.at[:, pl.ds((w + _NW * i) * _CB, _CB)]

    for j in range(_NIT):
        pltpu.async_copy(table_hbm.at[idx_v.at[j]], gbuf.at[j], gsem.at[j])
    iota = lax.iota(jnp.int32, _L)

    for i in range(_NIT):
        p = i % 2
        blk = blk_v.at[p]
        if i >= 2:
            pltpu.make_async_copy(blk, out_slice(i - 2), osem.at[p]).wait()
        pltpu.make_async_copy(table_hbm.at[idx_v.at[i]], gbuf.at[i],
                              gsem.at[i]).wait()

        def row(r, carry, i=i, blk=blk):
            cols = jnp.broadcast_to(r, (_L,)).astype(jnp.int32)
            for q in range(_D // _L):
                val = gbuf[i, r, pl.ds(q * _L, _L)]
                plsc.store_scatter(blk, [iota + (q * _L), cols], val)
            return carry

        lax.fori_loop(0, _CB, row, 0)
        pltpu.async_copy(blk, out_slice(i), osem.at[p])

    for j in (_NIT - 2, _NIT - 1):
        pltpu.make_async_copy(blk_v.at[j % 2], out_slice(j),
                              osem.at[j % 2]).wait()


@jax.jit
def _gather_sc(sid, table128):
    kern = pl.kernel(
        _gather_body,
        out_type=jax.ShapeDtypeStruct((_D, _B), jnp.float32),
        mesh=plsc.VectorSubcoreMesh(core_axis_name="c", subcore_axis_name="s"),
        compiler_params=pltpu.CompilerParams(needs_layout_passes=False),
        scratch_types=[
            pltpu.VMEM((_NIT, _CB), jnp.int32),
            pltpu.VMEM((_NIT, _CB, 2 * _D), jnp.float32),
            pltpu.VMEM((2, _D, _CB), jnp.float32),
            pltpu.SemaphoreType.DMA,
            pltpu.SemaphoreType.DMA((_NIT,)),
            pltpu.SemaphoreType.DMA((2,)),
        ],
    )
    return kern(sid, table128)


def _assemble_body(base_ref, feats_ref, wt_ref, bt_ref, out_ref):
    out_ref[pl.ds(0, _D), :] = base_ref[...]
    for f in range(_NF):
        fv = feats_ref[pl.ds(f, 1), :]
        wc = wt_ref[:, pl.ds(f, 1)]
        bc = bt_ref[:, pl.ds(f, 1)]
        out_ref[pl.ds((f + 1) * _D, _D), :] = wc * fv + bc


@jax.jit
def _assemble_tc(base2, feats, wt, bt):
    return pl.pallas_call(
        _assemble_body,
        grid=(_B // _TCB,),
        in_specs=[
            pl.BlockSpec((_D, _TCB), lambda j: (0, j)),
            pl.BlockSpec((_NF, _TCB), lambda j: (0, j)),
            pl.BlockSpec((_D, _NF), lambda j: (0, 0)),
            pl.BlockSpec((_D, _NF), lambda j: (0, 0)),
        ],
        out_specs=pl.BlockSpec((_OUT_W, _TCB), lambda j: (0, j)),
        out_shape=jax.ShapeDtypeStruct((_OUT_W, _B), jnp.float32),
    )(base2, feats, wt, bt)


def kernel(student_id, table,
           feat_age, feat_gender, feat_ethnicity, feat_location, feat_gpa,
           feat_test_scores, feat_courses, feat_major, feat_attendance,
           feat_participation, feat_feedback, feat_study_habits,
           feat_social_activity, feat_stress_level,
           W_age, W_gender, W_ethnicity, W_location, W_gpa,
           W_test_scores, W_courses, W_major, W_attendance,
           W_participation, W_feedback, W_study_habits,
           W_social_activity, W_stress_level,
           b_age, b_gender, b_ethnicity, b_location, b_gpa,
           b_test_scores, b_courses, b_major, b_attendance,
           b_participation, b_feedback, b_study_habits,
           b_social_activity, b_stress_level):
    feats = jnp.stack([
        feat_age, feat_gender, feat_ethnicity, feat_location, feat_gpa,
        feat_test_scores, feat_courses, feat_major, feat_attendance,
        feat_participation, feat_feedback, feat_study_habits,
        feat_social_activity, feat_stress_level])
    wmat = jnp.concatenate([
        W_age, W_gender, W_ethnicity, W_location, W_gpa,
        W_test_scores, W_courses, W_major, W_attendance,
        W_participation, W_feedback, W_study_habits,
        W_social_activity, W_stress_level], axis=0)
    bmat = jnp.stack([
        b_age, b_gender, b_ethnicity, b_location, b_gpa,
        b_test_scores, b_courses, b_major, b_attendance,
        b_participation, b_feedback, b_study_habits,
        b_social_activity, b_stress_level])
    table128 = jnp.concatenate(
        [table, jnp.zeros((table.shape[0], _D), table.dtype)], axis=1)
    base2 = _gather_sc(student_id, table128)
    out2 = _assemble_tc(base2, feats, wmat.T, bmat.T)
    return out2.T


# speedup vs baseline: 1.2760x; 1.0599x over previous
import functools

import jax
import jax.numpy as jnp
from jax import lax
from jax.experimental import pallas as pl
from jax.experimental.pallas import tpu as pltpu
from jax.experimental.pallas import tpu_sc as plsc

_B = 16384
_D = 64
_NF = 14
_OUT_W = (_NF + 1) * _D
_NW = 32
_CB = 128
_NIT = _B // _CB // _NW
_L = 16
_TCB = 1024


def _gather_body(sid_hbm, table_hbm, out_hbm, idx_v, gbuf, blk_v,
                 psem, gsem, osem):
    c_ax = lax.axis_index("c")
    s_ax = lax.axis_index("s")
    w = s_ax * 2 + c_ax

    pro = []
    for j in range(_NIT):
        cj = w + _NW * j
        pro.append(pltpu.async_copy(sid_hbm.at[pl.ds(cj * _CB, _CB)],
                                    idx_v.at[j], psem))
    for cp in pro:
        cp.wait()

    def out_slice(i):
        return out_hbm.at[:, pl.ds((w + _NW * i) * _CB, _CB)]

    for j in range(_NIT):
        pltpu.async_copy(table_hbm.at[idx_v.at[j]], gbuf.at[j], gsem.at[j])
    iota = lax.iota(jnp.int32, _L)

    for i in range(_NIT):
        p = i % 2
        blk = blk_v.at[p]
        if i >= 2:
            pltpu.make_async_copy(blk, out_slice(i - 2), osem.at[p]).wait()
        pltpu.make_async_copy(table_hbm.at[idx_v.at[i]], gbuf.at[i],
                              gsem.at[i]).wait()

        def row(r, carry, i=i, blk=blk):
            cols = jnp.broadcast_to(r, (_L,)).astype(jnp.int32)
            for q in range(_D // _L):
                val = gbuf[i, r, pl.ds(q * _L, _L)]
                plsc.store_scatter(blk, [iota + (q * _L), cols], val)
            return carry

        lax.fori_loop(0, _CB, row, 0)
        pltpu.async_copy(blk, out_slice(i), osem.at[p])

    for j in (_NIT - 2, _NIT - 1):
        pltpu.make_async_copy(blk_v.at[j % 2], out_slice(j),
                              osem.at[j % 2]).wait()


@jax.jit
def _gather_sc(sid, table):
    kern = pl.kernel(
        _gather_body,
        out_type=jax.ShapeDtypeStruct((_D, _B), jnp.float32),
        mesh=plsc.VectorSubcoreMesh(core_axis_name="c", subcore_axis_name="s"),
        compiler_params=pltpu.CompilerParams(needs_layout_passes=False,
                                             use_tc_tiling_on_sc=False),
        scratch_types=[
            pltpu.VMEM((_NIT, _CB), jnp.int32),
            pltpu.VMEM((_NIT, _CB, _D), jnp.float32),
            pltpu.VMEM((2, _D, _CB), jnp.float32),
            pltpu.SemaphoreType.DMA,
            pltpu.SemaphoreType.DMA((_NIT,)),
            pltpu.SemaphoreType.DMA((2,)),
        ],
    )
    return kern(sid, table)


def _assemble_body(base_ref, feats_ref, wt_ref, bt_ref, out_ref):
    out_ref[pl.ds(0, _D), :] = base_ref[...]
    for f in range(_NF):
        fv = feats_ref[pl.ds(f, 1), :]
        wc = wt_ref[:, pl.ds(f, 1)]
        bc = bt_ref[:, pl.ds(f, 1)]
        out_ref[pl.ds((f + 1) * _D, _D), :] = wc * fv + bc


@jax.jit
def _assemble_tc(base2, feats, wt, bt):
    return pl.pallas_call(
        _assemble_body,
        grid=(_B // _TCB,),
        in_specs=[
            pl.BlockSpec((_D, _TCB), lambda j: (0, j)),
            pl.BlockSpec((_NF, _TCB), lambda j: (0, j)),
            pl.BlockSpec((_D, _NF), lambda j: (0, 0)),
            pl.BlockSpec((_D, _NF), lambda j: (0, 0)),
        ],
        out_specs=pl.BlockSpec((_OUT_W, _TCB), lambda j: (0, j)),
        out_shape=jax.ShapeDtypeStruct((_OUT_W, _B), jnp.float32),
    )(base2, feats, wt, bt)


def kernel(student_id, table,
           feat_age, feat_gender, feat_ethnicity, feat_location, feat_gpa,
           feat_test_scores, feat_courses, feat_major, feat_attendance,
           feat_participation, feat_feedback, feat_study_habits,
           feat_social_activity, feat_stress_level,
           W_age, W_gender, W_ethnicity, W_location, W_gpa,
           W_test_scores, W_courses, W_major, W_attendance,
           W_participation, W_feedback, W_study_habits,
           W_social_activity, W_stress_level,
           b_age, b_gender, b_ethnicity, b_location, b_gpa,
           b_test_scores, b_courses, b_major, b_attendance,
           b_participation, b_feedback, b_study_habits,
           b_social_activity, b_stress_level):
    feats = jnp.stack([
        feat_age, feat_gender, feat_ethnicity, feat_location, feat_gpa,
        feat_test_scores, feat_courses, feat_major, feat_attendance,
        feat_participation, feat_feedback, feat_study_habits,
        feat_social_activity, feat_stress_level])
    wmat = jnp.concatenate([
        W_age, W_gender, W_ethnicity, W_location, W_gpa,
        W_test_scores, W_courses, W_major, W_attendance,
        W_participation, W_feedback, W_study_habits,
        W_social_activity, W_stress_level], axis=0)
    bmat = jnp.stack([
        b_age, b_gender, b_ethnicity, b_location, b_gpa,
        b_test_scores, b_courses, b_major, b_attendance,
        b_participation, b_feedback, b_study_habits,
        b_social_activity, b_stress_level])
    base2 = _gather_sc(student_id, table)
    out2 = _assemble_tc(base2, feats, wmat.T, bmat.T)
    return out2.T


# speedup vs baseline: 1.3444x; 1.0536x over previous
import functools

import jax
import jax.numpy as jnp
from jax import lax
from jax.experimental import pallas as pl
from jax.experimental.pallas import tpu as pltpu
from jax.experimental.pallas import tpu_sc as plsc

_B = 16384
_D = 64
_NF = 14
_OUT_W = (_NF + 1) * _D
_NW = 32
_CB = 128
_NIT = _B // _CB // _NW
_L = 16
_TCB = 1024


def _gather_body(sid_hbm, table_hbm, out_hbm, idx_v, gbuf, blk_v,
                 psem, gsem, osem):
    c_ax = lax.axis_index("c")
    s_ax = lax.axis_index("s")
    w = s_ax * 2 + c_ax

    pro = []
    for j in range(_NIT):
        cj = w + _NW * j
        pro.append(pltpu.async_copy(sid_hbm.at[pl.ds(cj * _CB, _CB)],
                                    idx_v.at[j], psem))
    for cp in pro:
        cp.wait()

    def out_slice(i):
        return out_hbm.at[:, pl.ds((w + _NW * i) * _CB, _CB)]

    for j in range(_NIT):
        pltpu.async_copy(table_hbm.at[idx_v.at[j]], gbuf.at[j], gsem.at[j])
    iota = lax.iota(jnp.int32, _L)

    for i in range(_NIT):
        p = i % 2
        blk = blk_v.at[p]
        if i >= 2:
            pltpu.make_async_copy(blk, out_slice(i - 2), osem.at[p]).wait()
        pltpu.make_async_copy(table_hbm.at[idx_v.at[i]], gbuf.at[i],
                              gsem.at[i]).wait()

        def row(r, carry, i=i, blk=blk):
            cols = jnp.broadcast_to(r, (_L,)).astype(jnp.int32)
            for q in range(_D // _L):
                val = gbuf[i, r, pl.ds(q * _L, _L)]
                plsc.store_scatter(blk, [iota + (q * _L), cols], val)
            return carry

        lax.fori_loop(0, _CB, row, 0)
        pltpu.async_copy(blk, out_slice(i), osem.at[p])

    for j in (_NIT - 2, _NIT - 1):
        pltpu.make_async_copy(blk_v.at[j % 2], out_slice(j),
                              osem.at[j % 2]).wait()


@jax.jit
def _gather_sc(sid, table128):
    kern = pl.kernel(
        _gather_body,
        out_type=jax.ShapeDtypeStruct((_D, _B), jnp.float32),
        mesh=plsc.VectorSubcoreMesh(core_axis_name="c", subcore_axis_name="s"),
        compiler_params=pltpu.CompilerParams(needs_layout_passes=False),
        scratch_types=[
            pltpu.VMEM((_NIT, _CB), jnp.int32),
            pltpu.VMEM((_NIT, _CB, 2 * _D), jnp.float32),
            pltpu.VMEM((2, _D, _CB), jnp.float32),
            pltpu.SemaphoreType.DMA,
            pltpu.SemaphoreType.DMA((_NIT,)),
            pltpu.SemaphoreType.DMA((2,)),
        ],
    )
    return kern(sid, table128)


def _assemble_body(base_ref, feats_ref, wt_ref, bt_ref, out_ref):
    out_ref[pl.ds(0, _D), :] = base_ref[...]
    for f in range(_NF):
        fv = feats_ref[pl.ds(f, 1), :]
        wc = wt_ref[:, pl.ds(f, 1)]
        bc = bt_ref[:, pl.ds(f, 1)]
        out_ref[pl.ds((f + 1) * _D, _D), :] = wc * fv + bc


@jax.jit
def _assemble_tc(base2, feats, wt, bt):
    return pl.pallas_call(
        _assemble_body,
        grid=(_B // _TCB,),
        in_specs=[
            pl.BlockSpec((_D, _TCB), lambda j: (0, j)),
            pl.BlockSpec((_NF, _TCB), lambda j: (0, j)),
            pl.BlockSpec((_D, _NF), lambda j: (0, 0)),
            pl.BlockSpec((_D, _NF), lambda j: (0, 0)),
        ],
        out_specs=pl.BlockSpec((_OUT_W, _TCB), lambda j: (0, j)),
        out_shape=jax.ShapeDtypeStruct((_OUT_W, _B), jnp.float32),
    )(base2, feats, wt, bt)


def kernel(student_id, table,
           feat_age, feat_gender, feat_ethnicity, feat_location, feat_gpa,
           feat_test_scores, feat_courses, feat_major, feat_attendance,
           feat_participation, feat_feedback, feat_study_habits,
           feat_social_activity, feat_stress_level,
           W_age, W_gender, W_ethnicity, W_location, W_gpa,
           W_test_scores, W_courses, W_major, W_attendance,
           W_participation, W_feedback, W_study_habits,
           W_social_activity, W_stress_level,
           b_age, b_gender, b_ethnicity, b_location, b_gpa,
           b_test_scores, b_courses, b_major, b_attendance,
           b_participation, b_feedback, b_study_habits,
           b_social_activity, b_stress_level):
    feats = jnp.stack([
        feat_age, feat_gender, feat_ethnicity, feat_location, feat_gpa,
        feat_test_scores, feat_courses, feat_major, feat_attendance,
        feat_participation, feat_feedback, feat_study_habits,
        feat_social_activity, feat_stress_level])
    wmat = jnp.concatenate([
        W_age, W_gender, W_ethnicity, W_location, W_gpa,
        W_test_scores, W_courses, W_major, W_attendance,
        W_participation, W_feedback, W_study_habits,
        W_social_activity, W_stress_level], axis=0)
    bmat = jnp.stack([
        b_age, b_gender, b_ethnicity, b_location, b_gpa,
        b_test_scores, b_courses, b_major, b_attendance,
        b_participation, b_feedback, b_study_habits,
        b_social_activity, b_stress_level])
    table128 = jnp.concatenate(
        [table, jnp.zeros((table.shape[0], _D), table.dtype)], axis=1)
    base2 = _gather_sc(student_id, table128)
    out2 = _assemble_tc(base2, feats, wmat.T, bmat.T)
    return out2.T


# speedup vs baseline: 1.3464x; 1.0015x over previous
import jax
import jax.numpy as jnp
from jax import lax
from jax.experimental import pallas as pl
from jax.experimental.pallas import tpu as pltpu
from jax.experimental.pallas import tpu_sc as plsc

_B = 16384
_D = 64
_NF = 14
_OUT_W = (_NF + 1) * _D
_NW = 32
_CB = 128
_NIT = _B // _CB // _NW
_L = 16
_TCB = 1024


def _gather_body(sid_hbm, table_hbm, out_hbm, idx_v, gbuf, blk_v,
                 psem, gsem, osem):
    c_ax = lax.axis_index("c")
    s_ax = lax.axis_index("s")
    w = s_ax * 2 + c_ax

    pro = []
    for j in range(_NIT):
        cj = w + _NW * j
        pro.append(pltpu.async_copy(sid_hbm.at[pl.ds(cj * _CB, _CB)],
                                    idx_v.at[j], psem))
    for cp in pro:
        cp.wait()

    def out_slice(i):
        return out_hbm.at[:, pl.ds((w + _NW * i) * _CB, _CB)]

    for j in range(_NIT):
        pltpu.async_copy(table_hbm.at[idx_v.at[j]], gbuf.at[j], gsem.at[j])
    iota = lax.iota(jnp.int32, _L)

    for i in range(_NIT):
        p = i % 2
        blk = blk_v.at[p]
        if i >= 2:
            pltpu.make_async_copy(blk, out_slice(i - 2), osem.at[p]).wait()
        pltpu.make_async_copy(table_hbm.at[idx_v.at[i]], gbuf.at[i],
                              gsem.at[i]).wait()

        def row(r, carry, i=i, blk=blk):
            cols = jnp.broadcast_to(r, (_L,)).astype(jnp.int32)
            for q in range(_D // _L):
                val = gbuf[i, r, pl.ds(q * _L, _L)]
                plsc.store_scatter(blk, [iota + (q * _L), cols], val)
            return carry

        lax.fori_loop(0, _CB, row, 0)
        pltpu.async_copy(blk, out_slice(i), osem.at[p])

    for j in (_NIT - 2, _NIT - 1):
        pltpu.make_async_copy(blk_v.at[j % 2], out_slice(j),
                              osem.at[j % 2]).wait()


@jax.jit
def _gather_sc(sid, table128):
    kern = pl.kernel(
        _gather_body,
        out_type=jax.ShapeDtypeStruct((_D, _B), jnp.float32),
        mesh=plsc.VectorSubcoreMesh(core_axis_name="c", subcore_axis_name="s"),
        compiler_params=pltpu.CompilerParams(needs_layout_passes=False),
        scratch_types=[
            pltpu.VMEM((_NIT, _CB), jnp.int32),
            pltpu.VMEM((_NIT, _CB, 2 * _D), jnp.float32),
            pltpu.VMEM((2, _D, _CB), jnp.float32),
            pltpu.SemaphoreType.DMA,
            pltpu.SemaphoreType.DMA((_NIT,)),
            pltpu.SemaphoreType.DMA((2,)),
        ],
    )
    return kern(sid, table128)


def _assemble_body(base_ref, feats_ref, wt_ref, bt_ref, out_ref):
    out_ref[pl.ds(0, _D), :] = base_ref[...]
    for f in range(_NF):
        fv = feats_ref[pl.ds(f, 1), :]
        wc = wt_ref[:, pl.ds(f, 1)]
        bc = bt_ref[:, pl.ds(f, 1)]
        out_ref[pl.ds((f + 1) * _D, _D), :] = wc * fv + bc


@jax.jit
def _assemble_tc(base2, feats, wt, bt):
    return pl.pallas_call(
        _assemble_body,
        grid=(_B // _TCB,),
        in_specs=[
            pl.BlockSpec((_D, _TCB), lambda j: (0, j)),
            pl.BlockSpec((_NF, _TCB), lambda j: (0, j)),
            pl.BlockSpec((_D, _NF), lambda j: (0, 0)),
            pl.BlockSpec((_D, _NF), lambda j: (0, 0)),
        ],
        out_specs=pl.BlockSpec((_OUT_W, _TCB), lambda j: (0, j)),
        out_shape=jax.ShapeDtypeStruct((_OUT_W, _B), jnp.float32),
    )(base2, feats, wt, bt)


def kernel(student_id, table,
           feat_age, feat_gender, feat_ethnicity, feat_location, feat_gpa,
           feat_test_scores, feat_courses, feat_major, feat_attendance,
           feat_participation, feat_feedback, feat_study_habits,
           feat_social_activity, feat_stress_level,
           W_age, W_gender, W_ethnicity, W_location, W_gpa,
           W_test_scores, W_courses, W_major, W_attendance,
           W_participation, W_feedback, W_study_habits,
           W_social_activity, W_stress_level,
           b_age, b_gender, b_ethnicity, b_location, b_gpa,
           b_test_scores, b_courses, b_major, b_attendance,
           b_participation, b_feedback, b_study_habits,
           b_social_activity, b_stress_level):
    feats = jnp.stack([
        feat_age, feat_gender, feat_ethnicity, feat_location, feat_gpa,
        feat_test_scores, feat_courses, feat_major, feat_attendance,
        feat_participation, feat_feedback, feat_study_habits,
        feat_social_activity, feat_stress_level])
    wmat = jnp.concatenate([
        W_age, W_gender, W_ethnicity, W_location, W_gpa,
        W_test_scores, W_courses, W_major, W_attendance,
        W_participation, W_feedback, W_study_habits,
        W_social_activity, W_stress_level], axis=0)
    bmat = jnp.stack([
        b_age, b_gender, b_ethnicity, b_location, b_gpa,
        b_test_scores, b_courses, b_major, b_attendance,
        b_participation, b_feedback, b_study_habits,
        b_social_activity, b_stress_level])
    table128 = jnp.concatenate(
        [table, jnp.zeros((table.shape[0], _D), table.dtype)], axis=1)
    base2 = _gather_sc(student_id, table128)
    out2 = _assemble_tc(base2, feats, wmat.T, bmat.T)
    return out2.T
